# Initial kernel scaffold; baseline (speedup 1.0000x reference)
import jax
import jax.numpy as jnp
from jax.experimental import pallas as pl


def kernel(x, edge_index, W1, b1, W2, b2, Wc, bc):
    raise NotImplementedError("write your pallas kernel here")


# speedup vs baseline: 1.5328x; 1.5328x over previous
import functools

import jax
import jax.numpy as jnp
from jax.experimental import pallas as pl

N = 10
E_PAD = 96


def _fused_kernel(src_ref, dst_ref, dstr_ref, x_ref, w1_ref, b1_ref,
                  w2_ref, b2_ref, wc_ref, bc_ref, out_ref):
    f32 = jnp.float32
    src = src_ref[:, :]
    dst = dst_ref[:, :]
    dstr = dstr_ref[:, :]

    node_row = jax.lax.broadcasted_iota(jnp.int32, (E_PAD, N), 1)
    S = (src == node_row).astype(f32)
    D = (dst == node_row).astype(f32)
    node_col = jax.lax.broadcasted_iota(jnp.int32, (N, E_PAD), 0)
    Dt = (dstr == node_col).astype(f32)

    deg = 1.0 + jnp.sum(D, axis=0, keepdims=True)
    dis = jax.lax.rsqrt(deg)

    dis_src = jnp.sum(S * dis, axis=1, keepdims=True)
    dis_dst = jnp.sum(D * dis, axis=1, keepdims=True)
    norm = dis_src * dis_dst

    A = jnp.dot(Dt, S * norm, preferred_element_type=f32)
    eye = (jax.lax.broadcasted_iota(jnp.int32, (N, N), 0)
           == jax.lax.broadcasted_iota(jnp.int32, (N, N), 1)).astype(f32)
    A = A + eye * (1.0 / deg)

    xw = jnp.dot(x_ref[:, :], w1_ref[:, :], preferred_element_type=f32)
    h1 = jnp.maximum(jnp.dot(A, xw, preferred_element_type=f32)
                     + b1_ref[:, :], 0.0)
    hw = jnp.dot(h1, w2_ref[:, :], preferred_element_type=f32)
    h2 = jnp.maximum(jnp.dot(A, hw, preferred_element_type=f32)
                     + b2_ref[:, :], 0.0)

    logits = bc_ref[:, :]
    for n in range(N):
        logits = logits + jnp.dot(h2[n:n + 1, :], wc_ref[n],
                                  preferred_element_type=f32)

    m = jnp.max(logits, axis=1, keepdims=True)
    p = jnp.exp(logits - m)
    out_ref[:, :] = p / jnp.sum(p, axis=1, keepdims=True)


@jax.jit
def kernel(x, edge_index, W1, b1, W2, b2, Wc, bc):
    E = edge_index.shape[1]
    ei = edge_index.astype(jnp.int32)
    pad = jnp.full((2, E_PAD - E), -1, dtype=jnp.int32)
    ei = jnp.concatenate([ei, pad], axis=1)
    src = ei[0].reshape(E_PAD, 1)
    dst = ei[1].reshape(E_PAD, 1)
    dstr = ei[1].reshape(1, E_PAD)
    hid = W1.shape[1]
    ncls = Wc.shape[1]
    out = pl.pallas_call(
        _fused_kernel,
        out_shape=jax.ShapeDtypeStruct((1, ncls), jnp.float32),
    )(src, dst, dstr, x, W1, b1.reshape(1, hid),
      W2, b2.reshape(1, hid), Wc.reshape(N, hid, ncls), bc.reshape(1, ncls))
    return out


# speedup vs baseline: 1.5385x; 1.0037x over previous
import jax
import jax.numpy as jnp
from jax.experimental import pallas as pl
from jax.experimental.pallas import tpu as pltpu

N = 10
E_PAD = 96


def _fused_kernel(src_ref, dst_ref, dstr_ref, x_ref, b1_ref, b2_ref, bc_ref,
                  w1_hbm, w2_hbm, wc_hbm, out_ref,
                  w1_v, w2_v, wc_v, sem1, sem2, semc):
    f32 = jnp.float32
    cp1 = pltpu.make_async_copy(w1_hbm, w1_v, sem1)
    cp2 = pltpu.make_async_copy(w2_hbm, w2_v, sem2)
    cpc = pltpu.make_async_copy(wc_hbm, wc_v, semc)
    cp1.start()
    cp2.start()
    cpc.start()

    src = src_ref[:, :]
    dst = dst_ref[:, :]
    dstr = dstr_ref[:, :]
    node_row = jax.lax.broadcasted_iota(jnp.int32, (E_PAD, N), 1)
    S = (src == node_row).astype(f32)
    D = (dst == node_row).astype(f32)
    node_col = jax.lax.broadcasted_iota(jnp.int32, (N, E_PAD), 0)
    Dt = (dstr == node_col).astype(f32)

    deg = 1.0 + jnp.sum(D, axis=0, keepdims=True)
    dis = jax.lax.rsqrt(deg)
    dis_src = jnp.sum(S * dis, axis=1, keepdims=True)
    dis_dst = jnp.sum(D * dis, axis=1, keepdims=True)
    norm = dis_src * dis_dst
    A = jnp.dot(Dt, S * norm, preferred_element_type=f32)
    eye = (jax.lax.broadcasted_iota(jnp.int32, (N, N), 0)
           == jax.lax.broadcasted_iota(jnp.int32, (N, N), 1)).astype(f32)
    A = A + eye * (1.0 / deg)

    cp1.wait()
    xw = jnp.dot(x_ref[:, :], w1_v[:, :], preferred_element_type=f32)
    h1 = jnp.maximum(jnp.dot(A, xw, preferred_element_type=f32)
                     + b1_ref[:, :], 0.0)
    cp2.wait()
    hw = jnp.dot(h1, w2_v[:, :], preferred_element_type=f32)
    h2 = jnp.maximum(jnp.dot(A, hw, preferred_element_type=f32)
                     + b2_ref[:, :], 0.0)
    cpc.wait()
    logits = bc_ref[:, :]
    for n in range(N):
        logits = logits + jnp.dot(h2[n:n + 1, :], wc_v[n],
                                  preferred_element_type=f32)
    m = jnp.max(logits, axis=1, keepdims=True)
    p = jnp.exp(logits - m)
    out_ref[:, :] = p / jnp.sum(p, axis=1, keepdims=True)


@jax.jit
def kernel(x, edge_index, W1, b1, W2, b2, Wc, bc):
    E = edge_index.shape[1]
    ei = edge_index.astype(jnp.int32)
    pad = jnp.full((2, E_PAD - E), -1, dtype=jnp.int32)
    ei = jnp.concatenate([ei, pad], axis=1)
    src = ei[0].reshape(E_PAD, 1)
    dst = ei[1].reshape(E_PAD, 1)
    dstr = ei[1].reshape(1, E_PAD)
    inf, hid = W1.shape
    ncls = Wc.shape[1]
    wc3 = Wc.reshape(N, hid, ncls)
    vmem = pl.BlockSpec(memory_space=pltpu.MemorySpace.VMEM)
    hbm = pl.BlockSpec(memory_space=pltpu.MemorySpace.HBM)
    out = pl.pallas_call(
        _fused_kernel,
        out_shape=jax.ShapeDtypeStruct((1, ncls), jnp.float32),
        in_specs=[vmem, vmem, vmem, vmem, vmem, vmem, vmem, hbm, hbm, hbm],
        out_specs=vmem,
        scratch_shapes=[
            pltpu.VMEM((inf, hid), jnp.float32),
            pltpu.VMEM((hid, hid), jnp.float32),
            pltpu.VMEM((N, hid, ncls), jnp.float32),
            pltpu.SemaphoreType.DMA,
            pltpu.SemaphoreType.DMA,
            pltpu.SemaphoreType.DMA,
        ],
    )(src, dst, dstr, x, b1.reshape(1, hid), b2.reshape(1, hid),
      bc.reshape(1, ncls), W1, W2, wc3)
    return out
